# Initial kernel scaffold; baseline (speedup 1.0000x reference)
import jax
import jax.numpy as jnp
from jax.experimental import pallas as pl


def kernel(obs, state, W1, b1, W2, b2):
    raise NotImplementedError("write your pallas kernel here")


# speedup vs baseline: 3.8785x; 3.8785x over previous
import functools

import jax
import jax.numpy as jnp
from jax.experimental import pallas as pl

B, V, F, D_OBS, D_STATE, HID, K = 256, 128, 64, 256, 512, 1024, 3


def _state_proj_kernel(state_ref, w1s_ref, b1_ref, hs_ref):
    hs_ref[...] = (
        jnp.dot(state_ref[...], w1s_ref[...], preferred_element_type=jnp.float32)
        + b1_ref[...]
    )


def _adj_kernel(obs_ref, hs_ref, w1o_ref, w2_ref, b2_ref,
                sm_ref, adj_ref, ent_ref):
    h = jnp.dot(obs_ref[...], w1o_ref[...], preferred_element_type=jnp.float32)
    h = jax.nn.relu(h + hs_ref[...])
    logits = jnp.dot(h, w2_ref[...], preferred_element_type=jnp.float32)
    logits = logits + b2_ref[...]

    m = jnp.max(logits, axis=0, keepdims=True)
    e = jnp.exp(logits - m)
    s = jnp.sum(e, axis=0, keepdims=True)
    sm = e / s
    logp = (logits - m) - jnp.log(s)
    sm_ref[...] = sm

    ent = -jnp.sum(sm * logp, axis=0, keepdims=True)
    ent_ref[...] = jnp.sum(ent, axis=1, keepdims=True) / F

    iota = jax.lax.broadcasted_iota(jnp.int32, (V, F), 0)
    v0 = jnp.max(sm, axis=0, keepdims=True)
    i0 = jnp.min(jnp.where(sm == v0, iota, V), axis=0, keepdims=True)
    sm1 = jnp.where(iota == i0, -1.0, sm)
    v1 = jnp.max(sm1, axis=0, keepdims=True)
    i1 = jnp.min(jnp.where(sm1 == v1, iota, V), axis=0, keepdims=True)
    sm2 = jnp.where(iota == i1, -1.0, sm1)
    v2 = jnp.max(sm2, axis=0, keepdims=True)
    i2 = jnp.min(jnp.where(sm2 == v2, iota, V), axis=0, keepdims=True)

    p3 = v0 * v0 * v0
    p2 = 3.0 * v1 * v2 * (v1 + v2)
    p1 = 6.0 * v0 * v1 * v2
    c3 = (p3 > p2) & (p3 > p1)
    c2 = (p2 >= p3) & (p2 > p1)
    j1 = jnp.where(c3, i0, i1)
    j2 = jnp.where(c3 | c2, i0, i2)

    cond2 = (iota == i0) | (iota == j1) | (iota == j2)
    cond1 = sm > 0.01
    adj_ref[...] = (cond1 & cond2).astype(jnp.int32)


@jax.jit
def kernel(obs, state, W1, b1, W2, b2):
    w1o = W1[:D_OBS]
    w1s = W1[D_OBS:]
    hs = pl.pallas_call(
        _state_proj_kernel,
        out_shape=jax.ShapeDtypeStruct((B, HID), jnp.float32),
    )(state, w1s, b1.reshape(1, HID))

    grid = (B,)
    sm, adj, ent = pl.pallas_call(
        _adj_kernel,
        grid=grid,
        in_specs=[
            pl.BlockSpec((None, V, D_OBS), lambda b: (b, 0, 0)),
            pl.BlockSpec((None, 1, HID), lambda b: (b, 0, 0)),
            pl.BlockSpec((D_OBS, HID), lambda b: (0, 0)),
            pl.BlockSpec((HID, F), lambda b: (0, 0)),
            pl.BlockSpec((1, F), lambda b: (0, 0)),
        ],
        out_specs=[
            pl.BlockSpec((None, V, F), lambda b: (b, 0, 0)),
            pl.BlockSpec((None, V, F), lambda b: (b, 0, 0)),
            pl.BlockSpec((None, 1, 1), lambda b: (b, 0, 0)),
        ],
        out_shape=[
            jax.ShapeDtypeStruct((B, V, F), jnp.float32),
            jax.ShapeDtypeStruct((B, V, F), jnp.int32),
            jax.ShapeDtypeStruct((B, 1, 1), jnp.float32),
        ],
    )(obs, hs.reshape(B, 1, HID), w1o, W2, b2.reshape(1, F))
    return sm, adj, ent.reshape(B)


# speedup vs baseline: 7.2799x; 1.8770x over previous
import jax
import jax.numpy as jnp
from jax.experimental import pallas as pl

B, V, F, D_OBS, D_STATE, HID, K = 256, 128, 64, 256, 512, 1024, 3
BB = 4


def _adj_kernel(obs_ref, state_ref, w1o_ref, w1s_ref, b1_ref, w2_ref, b2_ref,
                sm_ref, adj_ref, ent_ref):
    hs = (
        jnp.dot(state_ref[...], w1s_ref[...], preferred_element_type=jnp.float32)
        + b1_ref[...]
    )
    obs2d = obs_ref[...].reshape(BB * V, D_OBS)
    h = jnp.dot(obs2d, w1o_ref[...], preferred_element_type=jnp.float32)
    h = jax.nn.relu(h.reshape(BB, V, HID) + hs[:, None, :])
    logits = jnp.dot(h.reshape(BB * V, HID), w2_ref[...],
                     preferred_element_type=jnp.float32)
    logits = (logits + b2_ref[...]).reshape(BB, V, F)

    m = jnp.max(logits, axis=1, keepdims=True)
    e = jnp.exp(logits - m)
    s = jnp.sum(e, axis=1, keepdims=True)
    sm = e / s
    logp = (logits - m) - jnp.log(s)
    sm_ref[...] = sm

    ent = -jnp.sum(sm * logp, axis=1, keepdims=True)
    ent_ref[...] = jnp.sum(ent, axis=2, keepdims=True) / F

    iota = jax.lax.broadcasted_iota(jnp.int32, (BB, V, F), 1)
    v0 = jnp.max(sm, axis=1, keepdims=True)
    i0 = jnp.min(jnp.where(sm == v0, iota, V), axis=1, keepdims=True)
    sm1 = jnp.where(iota == i0, -1.0, sm)
    v1 = jnp.max(sm1, axis=1, keepdims=True)
    i1 = jnp.min(jnp.where(sm1 == v1, iota, V), axis=1, keepdims=True)
    sm2 = jnp.where(iota == i1, -1.0, sm1)
    v2 = jnp.max(sm2, axis=1, keepdims=True)
    i2 = jnp.min(jnp.where(sm2 == v2, iota, V), axis=1, keepdims=True)

    p3 = v0 * v0 * v0
    p2 = 3.0 * v1 * v2 * (v1 + v2)
    p1 = 6.0 * v0 * v1 * v2
    c3 = (p3 > p2) & (p3 > p1)
    c2 = (p2 >= p3) & (p2 > p1)
    j1 = jnp.where(c3, i0, i1)
    j2 = jnp.where(c3 | c2, i0, i2)

    cond2 = (iota == i0) | (iota == j1) | (iota == j2)
    cond1 = sm > 0.01
    adj_ref[...] = (cond1 & cond2).astype(jnp.int32)


@jax.jit
def kernel(obs, state, W1, b1, W2, b2):
    w1o = W1[:D_OBS]
    w1s = W1[D_OBS:]
    grid = (B // BB,)
    sm, adj, ent = pl.pallas_call(
        _adj_kernel,
        grid=grid,
        in_specs=[
            pl.BlockSpec((BB, V, D_OBS), lambda b: (b, 0, 0)),
            pl.BlockSpec((None, BB, D_STATE), lambda b: (b, 0, 0)),
            pl.BlockSpec((D_OBS, HID), lambda b: (0, 0)),
            pl.BlockSpec((D_STATE, HID), lambda b: (0, 0)),
            pl.BlockSpec((1, HID), lambda b: (0, 0)),
            pl.BlockSpec((HID, F), lambda b: (0, 0)),
            pl.BlockSpec((1, F), lambda b: (0, 0)),
        ],
        out_specs=[
            pl.BlockSpec((BB, V, F), lambda b: (b, 0, 0)),
            pl.BlockSpec((BB, V, F), lambda b: (b, 0, 0)),
            pl.BlockSpec((BB, 1, 1), lambda b: (b, 0, 0)),
        ],
        out_shape=[
            jax.ShapeDtypeStruct((B, V, F), jnp.float32),
            jax.ShapeDtypeStruct((B, V, F), jnp.int32),
            jax.ShapeDtypeStruct((B, 1, 1), jnp.float32),
        ],
    )(obs, state.reshape(B // BB, BB, D_STATE), w1o, w1s,
      b1.reshape(1, HID), W2, b2.reshape(1, F))
    return sm, adj, ent.reshape(B)


# speedup vs baseline: 8.2905x; 1.1388x over previous
import jax
import jax.numpy as jnp
from jax.experimental import pallas as pl

B, V, F, D_OBS, D_STATE, HID, K = 256, 128, 64, 256, 512, 1024, 3
BB = 8


def _adj_kernel(obs_ref, state_ref, w1o_ref, w1s_ref, b1_ref, w2_ref, b2_ref,
                sm_ref, adj_ref, ent_ref):
    hs = (
        jnp.dot(state_ref[...], w1s_ref[...], preferred_element_type=jnp.float32)
        + b1_ref[...]
    )
    obs2d = obs_ref[...].reshape(BB * V, D_OBS)
    h = jnp.dot(obs2d, w1o_ref[...], preferred_element_type=jnp.float32)
    h = jax.nn.relu(h.reshape(BB, V, HID) + hs[:, None, :])
    logits = jnp.dot(h.reshape(BB * V, HID), w2_ref[...],
                     preferred_element_type=jnp.float32)
    logits = (logits + b2_ref[...]).reshape(BB, V, F)

    m = jnp.max(logits, axis=1, keepdims=True)
    e = jnp.exp(logits - m)
    s = jnp.sum(e, axis=1, keepdims=True)
    sm = e / s
    logp = (logits - m) - jnp.log(s)
    sm_ref[...] = sm

    ent = -jnp.sum(sm * logp, axis=1, keepdims=True)
    ent_ref[...] = jnp.sum(ent, axis=2, keepdims=True) / F

    iota = jax.lax.broadcasted_iota(jnp.int32, (BB, V, F), 1)
    v0 = jnp.max(sm, axis=1, keepdims=True)
    i0 = jnp.min(jnp.where(sm == v0, iota, V), axis=1, keepdims=True)
    sm1 = jnp.where(iota == i0, -1.0, sm)
    v1 = jnp.max(sm1, axis=1, keepdims=True)
    i1 = jnp.min(jnp.where(sm1 == v1, iota, V), axis=1, keepdims=True)
    sm2 = jnp.where(iota == i1, -1.0, sm1)
    v2 = jnp.max(sm2, axis=1, keepdims=True)
    i2 = jnp.min(jnp.where(sm2 == v2, iota, V), axis=1, keepdims=True)

    p3 = v0 * v0 * v0
    p2 = 3.0 * v1 * v2 * (v1 + v2)
    p1 = 6.0 * v0 * v1 * v2
    c3 = (p3 > p2) & (p3 > p1)
    c2 = (p2 >= p3) & (p2 > p1)
    j1 = jnp.where(c3, i0, i1)
    j2 = jnp.where(c3 | c2, i0, i2)

    cond2 = (iota == i0) | (iota == j1) | (iota == j2)
    cond1 = sm > 0.01
    adj_ref[...] = (cond1 & cond2).astype(jnp.int32)


@jax.jit
def kernel(obs, state, W1, b1, W2, b2):
    w1o = W1[:D_OBS]
    w1s = W1[D_OBS:]
    grid = (B // BB,)
    sm, adj, ent = pl.pallas_call(
        _adj_kernel,
        grid=grid,
        in_specs=[
            pl.BlockSpec((BB, V, D_OBS), lambda b: (b, 0, 0)),
            pl.BlockSpec((None, BB, D_STATE), lambda b: (b, 0, 0)),
            pl.BlockSpec((D_OBS, HID), lambda b: (0, 0)),
            pl.BlockSpec((D_STATE, HID), lambda b: (0, 0)),
            pl.BlockSpec((1, HID), lambda b: (0, 0)),
            pl.BlockSpec((HID, F), lambda b: (0, 0)),
            pl.BlockSpec((1, F), lambda b: (0, 0)),
        ],
        out_specs=[
            pl.BlockSpec((BB, V, F), lambda b: (b, 0, 0)),
            pl.BlockSpec((BB, V, F), lambda b: (b, 0, 0)),
            pl.BlockSpec((BB, 1, 1), lambda b: (b, 0, 0)),
        ],
        out_shape=[
            jax.ShapeDtypeStruct((B, V, F), jnp.float32),
            jax.ShapeDtypeStruct((B, V, F), jnp.int32),
            jax.ShapeDtypeStruct((B, 1, 1), jnp.float32),
        ],
    )(obs, state.reshape(B // BB, BB, D_STATE), w1o, w1s,
      b1.reshape(1, HID), W2, b2.reshape(1, F))
    return sm, adj, ent.reshape(B)


# speedup vs baseline: 9.3142x; 1.1235x over previous
import jax
import jax.numpy as jnp
from jax.experimental import pallas as pl

B, V, F, D_OBS, D_STATE, HID, K = 256, 128, 64, 256, 512, 1024, 3
BB = 8


def _adj_kernel(obs_ref, state_ref, w1o_ref, w1s_ref, b1_ref, w2_ref, b2_ref,
                sm_ref, adj_ref, ent_ref):
    hs = (
        jnp.dot(state_ref[...], w1s_ref[...], preferred_element_type=jnp.float32)
        + b1_ref[...]
    )
    obs2d = obs_ref[...].reshape(BB * V, D_OBS)
    h = jnp.dot(obs2d, w1o_ref[...], preferred_element_type=jnp.float32)
    h = jax.nn.relu(h.reshape(BB, V, HID) + hs[:, None, :])
    logits = jnp.dot(h.reshape(BB * V, HID), w2_ref[...],
                     preferred_element_type=jnp.float32)
    logits = (logits + b2_ref[...]).reshape(BB, V, F)

    G = BB // 2
    logits = jnp.concatenate([logits[:G], logits[G:]], axis=2)

    m = jnp.max(logits, axis=1, keepdims=True)
    e = jnp.exp(logits - m)
    s = jnp.sum(e, axis=1, keepdims=True)
    sm = e / s
    logp = (logits - m) - jnp.log(s)
    sm_ref[...] = jnp.concatenate([sm[:, :, :F], sm[:, :, F:]], axis=0)

    ent = -jnp.sum(sm * logp, axis=1, keepdims=True)
    ent_lo = jnp.sum(ent[:, :, :F], axis=2, keepdims=True) / F
    ent_hi = jnp.sum(ent[:, :, F:], axis=2, keepdims=True) / F
    ent_ref[...] = jnp.concatenate([ent_lo, ent_hi], axis=0)

    iota = jax.lax.broadcasted_iota(jnp.int32, (G, V, 2 * F), 1)
    v0 = jnp.max(sm, axis=1, keepdims=True)
    i0 = jnp.min(jnp.where(sm == v0, iota, V), axis=1, keepdims=True)
    sm1 = jnp.where(iota == i0, -1.0, sm)
    v1 = jnp.max(sm1, axis=1, keepdims=True)
    i1 = jnp.min(jnp.where(sm1 == v1, iota, V), axis=1, keepdims=True)
    sm2 = jnp.where(iota == i1, -1.0, sm1)
    v2 = jnp.max(sm2, axis=1, keepdims=True)
    i2 = jnp.min(jnp.where(sm2 == v2, iota, V), axis=1, keepdims=True)

    p3 = v0 * v0 * v0
    p2 = 3.0 * v1 * v2 * (v1 + v2)
    p1 = 6.0 * v0 * v1 * v2
    c3 = (p3 > p2) & (p3 > p1)
    c2 = (p2 >= p3) & (p2 > p1)
    j1 = jnp.where(c3, i0, i1)
    j2 = jnp.where(c3 | c2, i0, i2)

    cond2 = (iota == i0) | (iota == j1) | (iota == j2)
    cond1 = sm > 0.01
    adj = (cond1 & cond2).astype(jnp.int32)
    adj_ref[...] = jnp.concatenate([adj[:, :, :F], adj[:, :, F:]], axis=0)


@jax.jit
def kernel(obs, state, W1, b1, W2, b2):
    w1o = W1[:D_OBS]
    w1s = W1[D_OBS:]
    grid = (B // BB,)
    sm, adj, ent = pl.pallas_call(
        _adj_kernel,
        grid=grid,
        in_specs=[
            pl.BlockSpec((BB, V, D_OBS), lambda b: (b, 0, 0)),
            pl.BlockSpec((None, BB, D_STATE), lambda b: (b, 0, 0)),
            pl.BlockSpec((D_OBS, HID), lambda b: (0, 0)),
            pl.BlockSpec((D_STATE, HID), lambda b: (0, 0)),
            pl.BlockSpec((1, HID), lambda b: (0, 0)),
            pl.BlockSpec((HID, F), lambda b: (0, 0)),
            pl.BlockSpec((1, F), lambda b: (0, 0)),
        ],
        out_specs=[
            pl.BlockSpec((BB, V, F), lambda b: (b, 0, 0)),
            pl.BlockSpec((BB, V, F), lambda b: (b, 0, 0)),
            pl.BlockSpec((BB, 1, 1), lambda b: (b, 0, 0)),
        ],
        out_shape=[
            jax.ShapeDtypeStruct((B, V, F), jnp.float32),
            jax.ShapeDtypeStruct((B, V, F), jnp.int32),
            jax.ShapeDtypeStruct((B, 1, 1), jnp.float32),
        ],
    )(obs, state.reshape(B // BB, BB, D_STATE), w1o, w1s,
      b1.reshape(1, HID), W2, b2.reshape(1, F))
    return sm, adj, ent.reshape(B)


# speedup vs baseline: 9.8700x; 1.0597x over previous
import jax
import jax.numpy as jnp
from jax.experimental import pallas as pl
from jax.experimental.pallas import tpu as pltpu

B, V, F, D_OBS, D_STATE, HID, K = 256, 128, 64, 256, 512, 1024, 3
BB = 8
G = BB // 2


def _adj_kernel(obs_ref, state_ref, w1o_ref, w1s_ref, b1_ref, w2_ref, b2_ref,
                sm_ref, adj_ref, ent_ref, nhs_ref, hsw2_ref):
    i = pl.program_id(0)

    @pl.when(i == 0)
    def _precompute():
        hs = (
            jnp.dot(state_ref[...], w1s_ref[...],
                    preferred_element_type=jnp.float32)
            + b1_ref[...]
        )
        nhs_ref[...] = -hs
        hsw2_ref[...] = (
            jnp.dot(hs, w2_ref[...], preferred_element_type=jnp.float32)
            + b2_ref[...]
        )

    nhs = nhs_ref[pl.ds(i * BB, BB), :]
    hsw2 = hsw2_ref[pl.ds(i * BB, BB), :]

    obs2d = obs_ref[...].reshape(BB * V, D_OBS)
    h0 = jnp.dot(obs2d, w1o_ref[...], preferred_element_type=jnp.float32)
    h = jnp.maximum(h0.reshape(BB, V, HID), nhs[:, None, :])
    logits = jnp.dot(h.reshape(BB * V, HID), w2_ref[...],
                     preferred_element_type=jnp.float32)
    logits = logits.reshape(BB, V, F) + hsw2[:, None, :]

    logits = jnp.concatenate([logits[:G], logits[G:]], axis=2)

    m = jnp.max(logits, axis=1, keepdims=True)
    e = jnp.exp(logits - m)
    s = jnp.sum(e, axis=1, keepdims=True)
    sm = e / s
    logp = (logits - m) - jnp.log(s)
    sm_ref[...] = jnp.concatenate([sm[:, :, :F], sm[:, :, F:]], axis=0)

    ent = -jnp.sum(sm * logp, axis=1, keepdims=True)
    ent_lo = jnp.sum(ent[:, :, :F], axis=2, keepdims=True) / F
    ent_hi = jnp.sum(ent[:, :, F:], axis=2, keepdims=True) / F
    ent_ref[...] = jnp.concatenate([ent_lo, ent_hi], axis=0)

    iota = jax.lax.broadcasted_iota(jnp.int32, (G, V, 2 * F), 1)
    v0 = jnp.max(sm, axis=1, keepdims=True)
    i0 = jnp.min(jnp.where(sm == v0, iota, V), axis=1, keepdims=True)
    sm1 = jnp.where(iota == i0, -1.0, sm)
    v1 = jnp.max(sm1, axis=1, keepdims=True)
    i1 = jnp.min(jnp.where(sm1 == v1, iota, V), axis=1, keepdims=True)
    sm2 = jnp.where(iota == i1, -1.0, sm1)
    v2 = jnp.max(sm2, axis=1, keepdims=True)
    i2 = jnp.min(jnp.where(sm2 == v2, iota, V), axis=1, keepdims=True)

    p3 = v0 * v0 * v0
    p2 = 3.0 * v1 * v2 * (v1 + v2)
    p1 = 6.0 * v0 * v1 * v2
    c3 = (p3 > p2) & (p3 > p1)
    c2 = (p2 >= p3) & (p2 > p1)
    j1 = jnp.where(c3, i0, i1)
    j2 = jnp.where(c3 | c2, i0, i2)

    cond2 = (iota == i0) | (iota == j1) | (iota == j2)
    cond1 = sm > 0.01
    adj = (cond1 & cond2).astype(jnp.int32)
    adj_ref[...] = jnp.concatenate([adj[:, :, :F], adj[:, :, F:]], axis=0)


@jax.jit
def kernel(obs, state, W1, b1, W2, b2):
    w1o = W1[:D_OBS]
    w1s = W1[D_OBS:]
    grid = (B // BB,)
    sm, adj, ent = pl.pallas_call(
        _adj_kernel,
        grid=grid,
        in_specs=[
            pl.BlockSpec((BB, V, D_OBS), lambda b: (b, 0, 0)),
            pl.BlockSpec((B, D_STATE), lambda b: (0, 0)),
            pl.BlockSpec((D_OBS, HID), lambda b: (0, 0)),
            pl.BlockSpec((D_STATE, HID), lambda b: (0, 0)),
            pl.BlockSpec((1, HID), lambda b: (0, 0)),
            pl.BlockSpec((HID, F), lambda b: (0, 0)),
            pl.BlockSpec((1, F), lambda b: (0, 0)),
        ],
        out_specs=[
            pl.BlockSpec((BB, V, F), lambda b: (b, 0, 0)),
            pl.BlockSpec((BB, V, F), lambda b: (b, 0, 0)),
            pl.BlockSpec((BB, 1, 1), lambda b: (b, 0, 0)),
        ],
        out_shape=[
            jax.ShapeDtypeStruct((B, V, F), jnp.float32),
            jax.ShapeDtypeStruct((B, V, F), jnp.int32),
            jax.ShapeDtypeStruct((B, 1, 1), jnp.float32),
        ],
        scratch_shapes=[
            pltpu.VMEM((B, HID), jnp.float32),
            pltpu.VMEM((B, F), jnp.float32),
        ],
    )(obs, state, w1o, w1s, b1.reshape(1, HID), W2, b2.reshape(1, F))
    return sm, adj, ent.reshape(B)
